# baseline (speedup 1.0000x reference)
import functools

import jax
import jax.numpy as jnp
from jax import lax
from jax.experimental import pallas as pl
from jax.experimental.pallas import tpu as pltpu
from jax.experimental.pallas import tpu_sc as plsc

MAX_DIM = 100000
EMBED_DIM = 32
LEN_COORD = 26
K = 16

NC, NS, L = 2, 16, 16
NW = NC * NS
CHUNK_GROUPS = 49
CHUNK_ROWS = CHUNK_GROUPS * L
NCHUNKS = 4
ROWS_PER_W = NCHUNKS * CHUNK_ROWS

_NEG_INF = float("-inf")

_GDN = lax.GatherDimensionNumbers(
    offset_dims=(), collapsed_slice_dims=(0,), start_index_map=(0,))


def _permute(x, idx):
    return lax.gather(x, idx[:, None], _GDN, slice_sizes=(1,),
                      mode=lax.GatherScatterMode.PROMISE_IN_BOUNDS)


def _splat(x, i):
    return _permute(x, jnp.full((L,), i, jnp.int32))


def _rne_bf16(x):
    b = lax.bitcast_convert_type(x, jnp.int32)
    lsb = (b >> 16) & 1
    r = (b + 0x7FFF + lsb) & jnp.int32(-65536)
    return lax.bitcast_convert_type(r, jnp.float32)


def _stage1_body(emb, w, bidx, bw, vals_out, idx_out,
                 buf, w_v, bidx_v, bw_v,
                 stage_v, stage_i, sem):
    wid = lax.axis_index("c") * NS + lax.axis_index("s")
    base_row = jnp.minimum(wid * ROWS_PER_W, MAX_DIM - ROWS_PER_W)
    iota = jnp.arange(L, dtype=jnp.int32)

    pltpu.sync_copy(w, w_v)
    pltpu.sync_copy(bidx, bidx_v)
    pltpu.sync_copy(bw, bw_v)

    w_lo = _rne_bf16(w_v[pl.ds(0, L)])
    w_hi = _rne_bf16(w_v[pl.ds(L, L)])
    wrot = []
    for k in range(EMBED_DIM):
        idx = (k + iota) % EMBED_DIM
        idxm = idx % L
        lo = _permute(w_lo, idxm)
        hi = _permute(w_hi, idxm)
        wrot.append(jnp.where(idx < L, lo, hi))

    bi_lo = bidx_v[pl.ds(0, L)]
    bi_hi = bidx_v[pl.ds(L, L)]
    bw_lo = bw_v[pl.ds(0, L)]
    bw_hi = bw_v[pl.ds(L, L)]
    rowids = []
    copies = []
    win = 8 * EMBED_DIM
    for j in range(2 * L):
        src = bi_lo if j < L else bi_hi
        rid = jnp.max(jnp.where(iota == (j % L), src, jnp.int32(0)))
        rowids.append(rid)
        copies.append(pltpu.async_copy(
            emb.at[pl.ds(pl.multiple_of((rid & ~7) * EMBED_DIM, win), win)],
            buf.at[pl.ds(j * win, win)], sem))
    for cp in copies:
        cp.wait()
    bf_lo = jnp.zeros((L,), jnp.float32)
    bf_hi = jnp.zeros((L,), jnp.float32)
    for j in range(2 * L):
        off = j * win + (rowids[j] & 7) * EMBED_DIM
        wgt = _splat(bw_lo if j < L else bw_hi, j % L)
        bf_lo = bf_lo + buf[pl.ds(off, L)] * wgt
        bf_hi = bf_hi + buf[pl.ds(off + L, L)] * wgt

    def _allsum(x):
        for sh in (1, 2, 4, 8):
            x = x + _permute(x, iota ^ sh)
        return x

    base_t = _allsum(bf_lo * w_lo + bf_hi * w_hi)
    s_bw = _allsum(jnp.abs(bf_lo) * jnp.abs(w_lo)
                   + jnp.abs(bf_hi) * jnp.abs(w_hi))
    s_w = _allsum(jnp.abs(w_lo) + jnp.abs(w_hi))
    eps_scale = jnp.float32(2.0**-8 + 2.0**-14)

    cand_v = jnp.full((L,), _NEG_INF, jnp.float32)
    cand_i = jnp.zeros((L,), jnp.int32)
    thr = jnp.full((L,), _NEG_INF, jnp.float32)

    ddiag = [(k + iota) % EMBED_DIM for k in range(EMBED_DIM)]

    for c in range(NCHUNKS):
        start = (base_row + c * CHUNK_ROWS) * EMBED_DIM
        pltpu.sync_copy(emb.at[pl.ds(start, CHUNK_ROWS * EMBED_DIM)], buf)

        def body(g, carry, c=c):
            cv, ci, th = carry
            row0 = g * L
            fbase = (row0 + iota) * EMBED_DIM
            accs = [jnp.zeros((L,), jnp.float32) for _ in range(4)]
            gmx = [jnp.zeros((L,), jnp.float32) for _ in range(4)]
            for d in range(EMBED_DIM):
                g_d = plsc.load_gather(buf, [fbase + ddiag[d]])
                accs[d % 4] = accs[d % 4] + g_d * wrot[d]
                gmx[d % 4] = jnp.maximum(gmx[d % 4], jnp.abs(g_d))
            s_apx = base_t + ((accs[0] + accs[1]) + (accs[2] + accs[3]))
            gmax = jnp.maximum(jnp.maximum(gmx[0], gmx[1]),
                               jnp.maximum(gmx[2], gmx[3]))
            eps = (s_bw + gmax * s_w) * eps_scale

            def exact(cv, ci, th):
                for r in range(L):
                    off = (row0 + r) * EMBED_DIM
                    lo = buf[pl.ds(off, L)]
                    hi = buf[pl.ds(off + L, L)]
                    buf[pl.ds(off, L)] = _rne_bf16(lo + bf_lo)
                    buf[pl.ds(off + L, L)] = _rne_bf16(hi + bf_hi)
                pacc = [jnp.zeros((L,), jnp.float32) for _ in range(4)]
                for d in range(EMBED_DIM):
                    q_d = plsc.load_gather(buf, [fbase + ddiag[d]])
                    pacc[d % 4] = pacc[d % 4] + q_d * wrot[d]
                pred = (pacc[0] + pacc[1]) + (pacc[2] + pacc[3])
                gidx = base_row + c * CHUNK_ROWS + row0 + iota
                nv, ni = plsc.sort_key_val(pred, gidx, descending=True)
                m = cv >= nv
                cv, ci = plsc.sort_key_val(jnp.where(m, cv, nv),
                                           jnp.where(m, ci, ni))
                return cv, ci, _splat(cv, 0)

            return lax.cond(jnp.any(s_apx + eps > th), exact,
                            lambda cv, ci, th: (cv, ci, th), cv, ci, th)

        cand_v, cand_i, thr = lax.fori_loop(0, CHUNK_GROUPS, body,
                                            (cand_v, cand_i, thr))

    stage_v[...] = cand_v
    stage_i[...] = cand_i
    pltpu.sync_copy(stage_v, vals_out.at[pl.ds(wid * L, L)])
    pltpu.sync_copy(stage_i, idx_out.at[pl.ds(wid * L, L)])


_stage1 = functools.partial(
    pl.kernel,
    out_type=(jax.ShapeDtypeStruct((NW * K,), jnp.float32),
              jax.ShapeDtypeStruct((NW * K,), jnp.int32)),
    mesh=plsc.VectorSubcoreMesh(core_axis_name="c", subcore_axis_name="s",
                                num_cores=NC, num_subcores=NS),
    compiler_params=pltpu.CompilerParams(needs_layout_passes=False,
                                         use_tc_tiling_on_sc=False),
    scratch_types=[
        pltpu.VMEM((CHUNK_ROWS * EMBED_DIM,), jnp.float32),
        pltpu.VMEM((EMBED_DIM,), jnp.float32),
        pltpu.VMEM((2 * L,), jnp.int32),
        pltpu.VMEM((2 * L,), jnp.float32),
        pltpu.VMEM((L,), jnp.float32),
        pltpu.VMEM((L,), jnp.int32),
        pltpu.SemaphoreType.DMA,
    ],
)(_stage1_body)


def _merge_body(vals_ref, idx_ref, str_ref, coord_ref, ti_ref, val_ref):
    v = vals_ref[...]
    gi = idx_ref[...]
    int_max = jnp.int32(2**31 - 1)
    kiota_r = lax.broadcasted_iota(jnp.int32, (1, K), 1)
    kiota_c = lax.broadcasted_iota(jnp.int32, (K, 1), 0)
    sel_v = jnp.zeros((1, K), jnp.float32)
    sel_i = jnp.zeros((K, 1), jnp.int32)
    for k in range(K):
        m = jnp.max(v, keepdims=True)
        a = jnp.min(jnp.where(v == m, gi, int_max), keepdims=True)
        sel_v = jnp.where(kiota_r == k, m, sel_v)
        sel_i = jnp.where(kiota_c == k, a, sel_i)
        v = jnp.where((v == m) & (gi == a), _NEG_INF, v)
    val_ref[...] = sel_v
    colio = lax.broadcasted_iota(jnp.int32, (K, LEN_COORD), 1)
    strb = jnp.broadcast_to(str_ref[...], (K, LEN_COORD))
    selb = jnp.broadcast_to(sel_i, (K, LEN_COORD))
    ti_ref[...] = jnp.where(colio == coord_ref[...], selb, strb)


_stage2 = pl.pallas_call(
    _merge_body,
    out_shape=(jax.ShapeDtypeStruct((K, LEN_COORD), jnp.int32),
               jax.ShapeDtypeStruct((1, K), jnp.float32)),
)


def kernel(emb_table, w_out, str_id, coordinate, num_samples):
    del num_samples
    w = w_out.reshape(EMBED_DIM)
    sid = str_id.reshape(LEN_COORD)
    coord = jnp.asarray(coordinate, jnp.int32)
    bidx = jnp.concatenate(
        [sid, jnp.zeros((2 * L - LEN_COORD,), sid.dtype)]).astype(jnp.int32)
    j = jnp.arange(2 * L, dtype=jnp.int32)
    bw = jnp.where((j < LEN_COORD) & (j != coord), 1.0, 0.0).astype(jnp.float32)
    cand_v, cand_i = _stage1(emb_table.reshape(-1), w, bidx, bw)
    top_inputs, values = _stage2(cand_v.reshape(1, NW * K),
                                 cand_i.reshape(1, NW * K),
                                 str_id.astype(jnp.int32),
                                 coord.reshape(1, 1))
    return top_inputs, values


# speedup vs baseline: 1.0890x; 1.0890x over previous
import functools

import jax
import jax.numpy as jnp
from jax import lax
from jax.experimental import pallas as pl
from jax.experimental.pallas import tpu as pltpu
from jax.experimental.pallas import tpu_sc as plsc

MAX_DIM = 100000
EMBED_DIM = 32
LEN_COORD = 26
K = 16

NC, NS, L = 2, 16, 16
NW = NC * NS
CHUNK_GROUPS = 49
CHUNK_ROWS = CHUNK_GROUPS * L
NCHUNKS = 4
ROWS_PER_W = NCHUNKS * CHUNK_ROWS

_NEG_INF = float("-inf")

_GDN = lax.GatherDimensionNumbers(
    offset_dims=(), collapsed_slice_dims=(0,), start_index_map=(0,))


def _permute(x, idx):
    return lax.gather(x, idx[:, None], _GDN, slice_sizes=(1,),
                      mode=lax.GatherScatterMode.PROMISE_IN_BOUNDS)


def _splat(x, i):
    return _permute(x, jnp.full((L,), i, jnp.int32))


def _rne_bf16(x):
    b = lax.bitcast_convert_type(x, jnp.int32)
    lsb = (b >> 16) & 1
    r = (b + 0x7FFF + lsb) & jnp.int32(-65536)
    return lax.bitcast_convert_type(r, jnp.float32)


def _stage1_body(emb, w, bidx, bw, vals_out, idx_out,
                 buf0, buf1, w_v, bidx_v, bw_v, rows_a, rows_b,
                 stage_v, stage_i, sem, semc0, semc1):
    wid = lax.axis_index("c") * NS + lax.axis_index("s")
    base_row = jnp.minimum(wid * ROWS_PER_W, MAX_DIM - ROWS_PER_W)
    iota = jnp.arange(L, dtype=jnp.int32)

    pltpu.sync_copy(w, w_v)
    pltpu.sync_copy(bidx, bidx_v)
    pltpu.sync_copy(bw, bw_v)

    bufs = [buf0, buf1]
    sems = [semc0, semc1]
    cp = pltpu.async_copy(emb.at[pl.ds(base_row, CHUNK_ROWS)], buf0, semc0)

    w_lo = _rne_bf16(w_v[pl.ds(0, L)])
    w_hi = _rne_bf16(w_v[pl.ds(L, L)])
    wrot = []
    for k in range(EMBED_DIM):
        idx = (k + iota) % EMBED_DIM
        idxm = idx % L
        lo = _permute(w_lo, idxm)
        hi = _permute(w_hi, idxm)
        wrot.append(jnp.where(idx < L, lo, hi))

    bi_lo = bidx_v[pl.ds(0, L)]
    bi_hi = bidx_v[pl.ds(L, L)]
    bw_lo = bw_v[pl.ds(0, L)]
    bw_hi = bw_v[pl.ds(L, L)]
    pltpu.async_copy(emb.at[bi_lo], rows_a, sem).wait()
    pltpu.async_copy(emb.at[bi_hi], rows_b, sem).wait()
    bf_lo = jnp.zeros((L,), jnp.float32)
    bf_hi = jnp.zeros((L,), jnp.float32)
    for j in range(2 * L):
        rows = rows_a if j < L else rows_b
        wgt = _splat(bw_lo if j < L else bw_hi, j % L)
        bf_lo = bf_lo + rows[j % L, pl.ds(0, L)] * wgt
        bf_hi = bf_hi + rows[j % L, pl.ds(L, L)] * wgt

    def _allsum(x):
        for sh in (1, 2, 4, 8):
            x = x + _permute(x, iota ^ sh)
        return x

    base_t = _allsum(bf_lo * w_lo + bf_hi * w_hi)
    s_bw = _allsum(jnp.abs(bf_lo) * jnp.abs(w_lo)
                   + jnp.abs(bf_hi) * jnp.abs(w_hi))
    s_w = _allsum(jnp.abs(w_lo) + jnp.abs(w_hi))
    eps_scale = jnp.float32(2.0**-8 + 2.0**-14)

    cand_v = jnp.full((L,), _NEG_INF, jnp.float32)
    cand_i = jnp.zeros((L,), jnp.int32)
    thr = jnp.full((L,), _NEG_INF, jnp.float32)

    ddiag = [(k + iota) % EMBED_DIM for k in range(EMBED_DIM)]

    for c in range(NCHUNKS):
        buf = bufs[c % 2]
        cp.wait()
        if c + 1 < NCHUNKS:
            start = base_row + (c + 1) * CHUNK_ROWS
            cp = pltpu.async_copy(emb.at[pl.ds(start, CHUNK_ROWS)],
                                  bufs[(c + 1) % 2], sems[(c + 1) % 2])

        def body(g, carry, c=c, buf=buf):
            cv, ci, th = carry
            row0 = g * L
            ridx = row0 + iota
            accs = [jnp.zeros((L,), jnp.float32) for _ in range(4)]
            gmx = [jnp.zeros((L,), jnp.float32) for _ in range(4)]
            for d in range(EMBED_DIM):
                g_d = plsc.load_gather(buf, [ridx, ddiag[d]])
                accs[d % 4] = accs[d % 4] + g_d * wrot[d]
                gmx[d % 4] = jnp.maximum(gmx[d % 4], jnp.abs(g_d))
            s_apx = base_t + ((accs[0] + accs[1]) + (accs[2] + accs[3]))
            gmax = jnp.maximum(jnp.maximum(gmx[0], gmx[1]),
                               jnp.maximum(gmx[2], gmx[3]))
            eps = (s_bw + gmax * s_w) * eps_scale

            def exact(cv, ci, th):
                for r in range(L):
                    lo = buf[row0 + r, pl.ds(0, L)]
                    hi = buf[row0 + r, pl.ds(L, L)]
                    buf[row0 + r, pl.ds(0, L)] = _rne_bf16(lo + bf_lo)
                    buf[row0 + r, pl.ds(L, L)] = _rne_bf16(hi + bf_hi)
                pacc = [jnp.zeros((L,), jnp.float32) for _ in range(4)]
                for d in range(EMBED_DIM):
                    q_d = plsc.load_gather(buf, [ridx, ddiag[d]])
                    pacc[d % 4] = pacc[d % 4] + q_d * wrot[d]
                pred = (pacc[0] + pacc[1]) + (pacc[2] + pacc[3])
                gidx = base_row + c * CHUNK_ROWS + row0 + iota
                nv, ni = plsc.sort_key_val(pred, gidx, descending=True)
                m = cv >= nv
                cv, ci = plsc.sort_key_val(jnp.where(m, cv, nv),
                                           jnp.where(m, ci, ni))
                return cv, ci, _splat(cv, 0)

            return lax.cond(jnp.any(s_apx + eps > th), exact,
                            lambda cv, ci, th: (cv, ci, th), cv, ci, th)

        cand_v, cand_i, thr = lax.fori_loop(0, CHUNK_GROUPS, body,
                                            (cand_v, cand_i, thr))

    stage_v[...] = cand_v
    stage_i[...] = cand_i
    pltpu.sync_copy(stage_v, vals_out.at[pl.ds(wid * L, L)])
    pltpu.sync_copy(stage_i, idx_out.at[pl.ds(wid * L, L)])


_stage1 = functools.partial(
    pl.kernel,
    out_type=(jax.ShapeDtypeStruct((NW * K,), jnp.float32),
              jax.ShapeDtypeStruct((NW * K,), jnp.int32)),
    mesh=plsc.VectorSubcoreMesh(core_axis_name="c", subcore_axis_name="s",
                                num_cores=NC, num_subcores=NS),
    compiler_params=pltpu.CompilerParams(needs_layout_passes=False,
                                         use_tc_tiling_on_sc=False),
    scratch_types=[
        pltpu.VMEM((CHUNK_ROWS, EMBED_DIM), jnp.float32),
        pltpu.VMEM((CHUNK_ROWS, EMBED_DIM), jnp.float32),
        pltpu.VMEM((EMBED_DIM,), jnp.float32),
        pltpu.VMEM((2 * L,), jnp.int32),
        pltpu.VMEM((2 * L,), jnp.float32),
        pltpu.VMEM((L, EMBED_DIM), jnp.float32),
        pltpu.VMEM((L, EMBED_DIM), jnp.float32),
        pltpu.VMEM((L,), jnp.float32),
        pltpu.VMEM((L,), jnp.int32),
        pltpu.SemaphoreType.DMA,
        pltpu.SemaphoreType.DMA,
        pltpu.SemaphoreType.DMA,
    ],
)(_stage1_body)


def _merge_body(vals_ref, idx_ref, str_ref, coord_ref, ti_ref, val_ref):
    v = vals_ref[...]
    gi = idx_ref[...]
    int_max = jnp.int32(2**31 - 1)
    kiota_r = lax.broadcasted_iota(jnp.int32, (1, K), 1)
    kiota_c = lax.broadcasted_iota(jnp.int32, (K, 1), 0)
    sel_v = jnp.zeros((1, K), jnp.float32)
    sel_i = jnp.zeros((K, 1), jnp.int32)
    for k in range(K):
        m = jnp.max(v, keepdims=True)
        a = jnp.min(jnp.where(v == m, gi, int_max), keepdims=True)
        sel_v = jnp.where(kiota_r == k, m, sel_v)
        sel_i = jnp.where(kiota_c == k, a, sel_i)
        v = jnp.where((v == m) & (gi == a), _NEG_INF, v)
    val_ref[...] = sel_v
    colio = lax.broadcasted_iota(jnp.int32, (K, LEN_COORD), 1)
    strb = jnp.broadcast_to(str_ref[...], (K, LEN_COORD))
    selb = jnp.broadcast_to(sel_i, (K, LEN_COORD))
    ti_ref[...] = jnp.where(colio == coord_ref[...], selb, strb)


_stage2 = pl.pallas_call(
    _merge_body,
    out_shape=(jax.ShapeDtypeStruct((K, LEN_COORD), jnp.int32),
               jax.ShapeDtypeStruct((1, K), jnp.float32)),
)


def kernel(emb_table, w_out, str_id, coordinate, num_samples):
    del num_samples
    w = w_out.reshape(EMBED_DIM)
    sid = str_id.reshape(LEN_COORD)
    coord = jnp.asarray(coordinate, jnp.int32)
    bidx = jnp.concatenate(
        [sid, jnp.zeros((2 * L - LEN_COORD,), sid.dtype)]).astype(jnp.int32)
    j = jnp.arange(2 * L, dtype=jnp.int32)
    bw = jnp.where((j < LEN_COORD) & (j != coord), 1.0, 0.0).astype(jnp.float32)
    cand_v, cand_i = _stage1(emb_table, w, bidx, bw)
    top_inputs, values = _stage2(cand_v.reshape(1, NW * K),
                                 cand_i.reshape(1, NW * K),
                                 str_id.astype(jnp.int32),
                                 coord.reshape(1, 1))
    return top_inputs, values
